# Initial kernel scaffold; baseline (speedup 1.0000x reference)
import jax
import jax.numpy as jnp
from jax.experimental import pallas as pl


def kernel(xyz, features, W1, b1, W2, b2, W3, b3):
    raise NotImplementedError("write your pallas kernel here")


# speedup vs baseline: 8.8425x; 8.8425x over previous
import functools

import jax
import jax.numpy as jnp
from jax import lax
from jax.experimental import pallas as pl
from jax.experimental.pallas import tpu as pltpu
from jax.experimental.pallas import tpu_sc as plsc

NPOINT = 1024
NSAMPLE = 32
RADIUS2 = 1.0
NROW = 8
LANES = 16
SBLK = 512


def _fps_body(xyz_ref, ox_ref, oy_ref, oz_ref, dists_ref):
    x = xyz_ref[0, 0]
    y = xyz_ref[0, 1]
    z = xyz_ref[0, 2]
    nrow, ncol = x.shape
    n = nrow * ncol
    lin = (lax.broadcasted_iota(jnp.int32, x.shape, 0) * ncol
           + lax.broadcasted_iota(jnp.int32, x.shape, 1))

    def body(i, far):
        sel = lin == far
        cx = jnp.sum(jnp.where(sel, x, 0.0))
        cy = jnp.sum(jnp.where(sel, y, 0.0))
        cz = jnp.sum(jnp.where(sel, z, 0.0))
        ox_ref[0, 0, i] = cx
        oy_ref[0, 0, i] = cy
        oz_ref[0, 0, i] = cz
        dx = x - cx
        dy = y - cy
        dz = z - cz
        d = dx * dx + dy * dy + dz * dz
        nd = jnp.minimum(dists_ref[...], d)
        dists_ref[...] = nd
        m = jnp.max(nd)
        return jnp.min(jnp.where(nd == m, lin, n)).astype(jnp.int32)

    dists_ref[...] = jnp.full(x.shape, 1e10, jnp.float32)
    lax.fori_loop(0, NPOINT, body, jnp.int32(0))


def _fps(xyz_r):
    b_, _, nrow, ncol = xyz_r.shape
    out_shape = [jax.ShapeDtypeStruct((b_, 1, NPOINT), jnp.float32)] * 3
    return pl.pallas_call(
        _fps_body,
        grid=(b_,),
        in_specs=[pl.BlockSpec((1, 3, nrow, ncol), lambda b: (b, 0, 0, 0))],
        out_specs=[pl.BlockSpec((1, 1, NPOINT), lambda b: (b, 0, 0),
                                memory_space=pltpu.SMEM)] * 3,
        out_shape=out_shape,
        scratch_shapes=[pltpu.VMEM((nrow, ncol), jnp.float32)],
    )(xyz_r)


def _make_ball_query(b_, n):
    nworkers = 32
    wpb = nworkers // b_
    cpw = NPOINT // wpb
    nchunk = n // LANES
    mesh = plsc.VectorSubcoreMesh(core_axis_name="c", subcore_axis_name="s")

    @functools.partial(
        pl.kernel,
        mesh=mesh,
        compiler_params=pltpu.CompilerParams(needs_layout_passes=False),
        out_type=jax.ShapeDtypeStruct((b_, 4 * NSAMPLE, NPOINT), jnp.float32),
        scratch_types=[
            pltpu.VMEM((n,), jnp.float32),
            pltpu.VMEM((n,), jnp.float32),
            pltpu.VMEM((n,), jnp.float32),
            pltpu.VMEM((n,), jnp.float32),
            pltpu.VMEM((cpw,), jnp.float32),
            pltpu.VMEM((cpw,), jnp.float32),
            pltpu.VMEM((cpw,), jnp.float32),
            pltpu.VMEM((NSAMPLE,), jnp.int32),
            pltpu.VMEM((4 * NSAMPLE, cpw), jnp.float32),
        ],
    )
    def bq(x_hbm, y_hbm, z_hbm, f_hbm, cx_hbm, cy_hbm, cz_hbm, out_hbm,
           x_v, y_v, z_v, f_v, cx_v, cy_v, cz_v, idx_v, out_v):
        wid = lax.axis_index("s") * 2 + lax.axis_index("c")
        b = wid // wpb
        k = wid % wpb
        pltpu.sync_copy(x_hbm.at[b], x_v)
        pltpu.sync_copy(y_hbm.at[b], y_v)
        pltpu.sync_copy(z_hbm.at[b], z_v)
        pltpu.sync_copy(f_hbm.at[b], f_v)
        pltpu.sync_copy(cx_hbm.at[b, pl.ds(k * cpw, cpw)], cx_v)
        pltpu.sync_copy(cy_hbm.at[b, pl.ds(k * cpw, cpw)], cy_v)
        pltpu.sync_copy(cz_hbm.at[b, pl.ds(k * cpw, cpw)], cz_v)
        lanes = lax.iota(jnp.int32, LANES)

        def per_centroid(ci, carry):
            splat_ci = jnp.full((LANES,), ci, jnp.int32)
            cxv = plsc.load_gather(cx_v, [splat_ci])
            cyv = plsc.load_gather(cy_v, [splat_ci])
            czv = plsc.load_gather(cz_v, [splat_ci])

            def w_cond(st):
                chunk, count = st
                return jnp.logical_and(count < NSAMPLE, chunk < nchunk)

            def w_body(st):
                chunk, count = st
                base = chunk * LANES
                xv = x_v[pl.ds(base, LANES)]
                yv = y_v[pl.ds(base, LANES)]
                zv = z_v[pl.ds(base, LANES)]
                dx = xv - cxv
                dy = yv - cyv
                dz = zv - czv
                d = dx * dx + dy * dy + dz * dz
                m = d <= RADIUS2
                cnt = jnp.max(plsc.all_reduce_population_count(m))

                @pl.when(cnt > 0)
                def _append():
                    rank = lax.cumsum(m.astype(jnp.int32))
                    pos = count + rank - 1
                    msel = jnp.logical_and(m, pos < NSAMPLE)
                    pos = jnp.clip(pos, 0, NSAMPLE - 1)
                    plsc.store_scatter(idx_v, [pos], base + lanes, mask=msel)

                return (chunk + 1, count + cnt)

            _, count = lax.while_loop(w_cond, w_body,
                                      (jnp.int32(0), jnp.int32(0)))
            head = idx_v[pl.ds(0, LANES)]
            first_s = jnp.max(jnp.where(lanes == 0, head,
                                        jnp.int32(-2147483648)))
            first = jnp.full((LANES,), first_s, jnp.int32)
            countv = jnp.full((LANES,), count, jnp.int32)
            for h in range(NSAMPLE // LANES):
                slots = lanes + LANES * h
                cur = idx_v[pl.ds(LANES * h, LANES)]
                idx_v[pl.ds(LANES * h, LANES)] = jnp.where(
                    slots < countv, cur, first)
            splat_c = jnp.full((LANES,), ci, jnp.int32)
            for h in range(NSAMPLE // LANES):
                iv = idx_v[pl.ds(LANES * h, LANES)]
                rows = (lanes + LANES * h) * 4
                gx = plsc.load_gather(x_v, [iv]) - cxv
                plsc.store_scatter(out_v, [rows, splat_c], gx)
                gy = plsc.load_gather(y_v, [iv]) - cyv
                plsc.store_scatter(out_v, [rows + 1, splat_c], gy)
                gz = plsc.load_gather(z_v, [iv]) - czv
                plsc.store_scatter(out_v, [rows + 2, splat_c], gz)
                gf = plsc.load_gather(f_v, [iv])
                plsc.store_scatter(out_v, [rows + 3, splat_c], gf)
            return carry

        lax.fori_loop(0, cpw, per_centroid, jnp.int32(0))
        pltpu.sync_copy(out_v, out_hbm.at[b, :, pl.ds(k * cpw, cpw)])

    return bq


def _mlp_body(x_ref, w1_ref, b1_ref, w2_ref, b2_ref, w3_ref, b3_ref, o_ref):
    w1 = w1_ref[...]
    b1 = b1_ref[...]
    w2 = w2_ref[...]
    b2 = b2_ref[...]
    w3 = w3_ref[...]
    b3 = b3_ref[...]

    def chain(xj):
        h = jnp.maximum(
            jnp.dot(w1, xj, preferred_element_type=jnp.float32) + b1, 0.0)
        h = jnp.maximum(
            jnp.dot(w2, h, preferred_element_type=jnp.float32) + b2, 0.0)
        return jnp.maximum(
            jnp.dot(w3, h, preferred_element_type=jnp.float32) + b3, 0.0)

    def body(j2, acc):
        blk = x_ref[0, pl.ds(j2 * 8, 8), :]
        acc = jnp.maximum(acc, chain(blk[0:4]))
        return jnp.maximum(acc, chain(blk[4:8]))

    o_ref[0] = lax.fori_loop(
        0, NSAMPLE // 2, body, jnp.zeros((64, SBLK), jnp.float32))


def _mlp(h0, w1, b1, w2, b2, w3, b3):
    b_ = h0.shape[0]
    return pl.pallas_call(
        _mlp_body,
        grid=(b_, NPOINT // SBLK),
        in_specs=[
            pl.BlockSpec((1, 4 * NSAMPLE, SBLK), lambda b, s: (b, 0, s)),
            pl.BlockSpec((32, 4), lambda b, s: (0, 0)),
            pl.BlockSpec((32, 1), lambda b, s: (0, 0)),
            pl.BlockSpec((32, 32), lambda b, s: (0, 0)),
            pl.BlockSpec((32, 1), lambda b, s: (0, 0)),
            pl.BlockSpec((64, 32), lambda b, s: (0, 0)),
            pl.BlockSpec((64, 1), lambda b, s: (0, 0)),
        ],
        out_specs=pl.BlockSpec((1, 64, SBLK), lambda b, s: (b, 0, s)),
        out_shape=jax.ShapeDtypeStruct((b_, 64, NPOINT), jnp.float32),
    )(h0, w1, b1, w2, b2, w3, b3)


def kernel(xyz, features, W1, b1, W2, b2, W3, b3):
    b_, n, _ = xyz.shape
    xyz_t = jnp.transpose(xyz, (0, 2, 1))
    xyz_r = xyz_t.reshape(b_, 3, NROW, n // NROW)
    ox, oy, oz = _fps(xyz_r)
    ox, oy, oz = ox[:, 0], oy[:, 0], oz[:, 0]
    new_xyz = jnp.stack([ox, oy, oz], axis=-1)
    h0 = _make_ball_query(b_, n)(
        xyz_t[:, 0], xyz_t[:, 1], xyz_t[:, 2], features[:, 0], ox, oy, oz)
    new_features = _mlp(h0, W1, b1.reshape(32, 1), W2, b2.reshape(32, 1),
                        W3, b3.reshape(64, 1))
    return new_xyz, new_features


# speedup vs baseline: 15.9992x; 1.8094x over previous
import functools

import jax
import jax.numpy as jnp
from jax import lax
from jax.experimental import pallas as pl
from jax.experimental.pallas import tpu as pltpu
from jax.experimental.pallas import tpu_sc as plsc

NPOINT = 1024
NSAMPLE = 32
RADIUS2 = 1.0
NROW = 8
LANES = 16
SBLK = 512


def _fps_body(xyz_ref, ox_ref, oy_ref, oz_ref, dists_ref):
    x = xyz_ref[0, 0]
    y = xyz_ref[0, 1]
    z = xyz_ref[0, 2]
    nrow, ncol = x.shape
    n = nrow * ncol
    lin = (lax.broadcasted_iota(jnp.int32, x.shape, 0) * ncol
           + lax.broadcasted_iota(jnp.int32, x.shape, 1))

    def body(i, far):
        sel = lin == far
        cx = jnp.sum(jnp.where(sel, x, 0.0))
        cy = jnp.sum(jnp.where(sel, y, 0.0))
        cz = jnp.sum(jnp.where(sel, z, 0.0))
        ox_ref[0, 0, i] = cx
        oy_ref[0, 0, i] = cy
        oz_ref[0, 0, i] = cz
        dx = x - cx
        dy = y - cy
        dz = z - cz
        d = dx * dx + dy * dy + dz * dz
        nd = jnp.minimum(dists_ref[...], d)
        dists_ref[...] = nd
        m = jnp.max(nd)
        return jnp.min(jnp.where(nd == m, lin, n)).astype(jnp.int32)

    dists_ref[...] = jnp.full(x.shape, 1e10, jnp.float32)
    lax.fori_loop(0, NPOINT, body, jnp.int32(0))


def _fps(xyz_r):
    b_, _, nrow, ncol = xyz_r.shape
    out_shape = [jax.ShapeDtypeStruct((b_, 1, NPOINT), jnp.float32)] * 3
    return pl.pallas_call(
        _fps_body,
        grid=(b_,),
        in_specs=[pl.BlockSpec((1, 3, nrow, ncol), lambda b: (b, 0, 0, 0))],
        out_specs=[pl.BlockSpec((1, 1, NPOINT), lambda b: (b, 0, 0),
                                memory_space=pltpu.SMEM)] * 3,
        out_shape=out_shape,
        scratch_shapes=[pltpu.VMEM((nrow, ncol), jnp.float32)],
    )(xyz_r)


UNROLL = 4


def _make_ball_query(b_, n):
    nworkers = 32
    wpb = nworkers // b_
    cpw = NPOINT // wpb
    nchunk = n // LANES
    mesh = plsc.VectorSubcoreMesh(core_axis_name="c", subcore_axis_name="s")

    @functools.partial(
        pl.kernel,
        mesh=mesh,
        compiler_params=pltpu.CompilerParams(needs_layout_passes=False),
        out_type=jax.ShapeDtypeStruct((b_, 4 * NSAMPLE, NPOINT), jnp.float32),
        scratch_types=[
            pltpu.VMEM((n,), jnp.float32),
            pltpu.VMEM((n,), jnp.float32),
            pltpu.VMEM((n,), jnp.float32),
            pltpu.VMEM((n,), jnp.float32),
            pltpu.VMEM((NPOINT,), jnp.float32),
            pltpu.VMEM((NPOINT,), jnp.float32),
            pltpu.VMEM((NPOINT,), jnp.float32),
            pltpu.VMEM((128,), jnp.int32),
            pltpu.VMEM((4 * NSAMPLE, cpw), jnp.float32),
        ],
    )
    def bq(x_hbm, y_hbm, z_hbm, f_hbm, cx_hbm, cy_hbm, cz_hbm, out_hbm,
           x_v, y_v, z_v, f_v, cx_v, cy_v, cz_v, idx_v, out_v):
        wid = lax.axis_index("s") * 2 + lax.axis_index("c")
        b = wid // wpb
        k = wid % wpb
        pltpu.sync_copy(x_hbm.at[b], x_v)
        pltpu.sync_copy(y_hbm.at[b], y_v)
        pltpu.sync_copy(z_hbm.at[b], z_v)
        pltpu.sync_copy(f_hbm.at[b], f_v)
        pltpu.sync_copy(cx_hbm.at[b], cx_v)
        pltpu.sync_copy(cy_hbm.at[b], cy_v)
        pltpu.sync_copy(cz_hbm.at[b], cz_v)
        lanes = lax.iota(jnp.int32, LANES)

        def per_centroid(ci, carry):
            splat_s = jnp.full((LANES,), ci * wpb + k, jnp.int32)
            cxv = plsc.load_gather(cx_v, [splat_s])
            cyv = plsc.load_gather(cy_v, [splat_s])
            czv = plsc.load_gather(cz_v, [splat_s])

            def w_cond(st):
                chunk, count = st
                return jnp.logical_and(count < NSAMPLE, chunk < nchunk)

            def w_body(st):
                chunk, count = st
                ms = []
                packed = jnp.zeros((LANES,), jnp.int32)
                for u in range(UNROLL):
                    base = (chunk + u) * LANES
                    xv = x_v[pl.ds(base, LANES)]
                    yv = y_v[pl.ds(base, LANES)]
                    zv = z_v[pl.ds(base, LANES)]
                    dx = xv - cxv
                    dy = yv - cyv
                    dz = zv - czv
                    d = dx * dx + dy * dy + dz * dz
                    m = d <= RADIUS2
                    ms.append(m)
                    cnt_u = plsc.all_reduce_population_count(m)
                    packed = packed + (cnt_u << (8 * u))
                tot = jnp.max(packed)

                @pl.when(tot > 0)
                def _append():
                    off = count
                    for u in range(UNROLL):
                        plsc.store_compressed(
                            idx_v.at[pl.ds(off, LANES)],
                            (chunk + u) * LANES + lanes, mask=ms[u])
                        off = off + ((tot >> (8 * u)) & 0xFF)

                total = ((tot & 0xFF) + ((tot >> 8) & 0xFF)
                         + ((tot >> 16) & 0xFF) + ((tot >> 24) & 0xFF))
                return (chunk + UNROLL, count + total)

            _, count = lax.while_loop(w_cond, w_body,
                                      (jnp.int32(0), jnp.int32(0)))
            head = idx_v[pl.ds(0, LANES)]
            first_s = jnp.max(jnp.where(lanes == 0, head,
                                        jnp.int32(-2147483648)))
            first = jnp.full((LANES,), first_s, jnp.int32)
            countv = jnp.full((LANES,), count, jnp.int32)
            for h in range(NSAMPLE // LANES):
                slots = lanes + LANES * h
                cur = idx_v[pl.ds(LANES * h, LANES)]
                idx_v[pl.ds(LANES * h, LANES)] = jnp.where(
                    slots < countv, cur, first)
            splat_c = jnp.full((LANES,), ci, jnp.int32)
            for h in range(NSAMPLE // LANES):
                iv = jnp.clip(idx_v[pl.ds(LANES * h, LANES)], 0, n - 1)
                rows = (lanes + LANES * h) * 4
                gx = plsc.load_gather(x_v, [iv]) - cxv
                plsc.store_scatter(out_v, [rows, splat_c], gx)
                gy = plsc.load_gather(y_v, [iv]) - cyv
                plsc.store_scatter(out_v, [rows + 1, splat_c], gy)
                gz = plsc.load_gather(z_v, [iv]) - czv
                plsc.store_scatter(out_v, [rows + 2, splat_c], gz)
                gf = plsc.load_gather(f_v, [iv])
                plsc.store_scatter(out_v, [rows + 3, splat_c], gf)
            return carry

        lax.fori_loop(0, cpw, per_centroid, jnp.int32(0))
        pltpu.sync_copy(out_v, out_hbm.at[b, :, pl.ds(k * cpw, cpw)])

    return bq


def _mlp_body(x_ref, w1_ref, b1_ref, w2_ref, b2_ref, w3_ref, b3_ref, o_ref):
    w1 = w1_ref[...]
    b1 = b1_ref[...]
    w2 = w2_ref[...]
    b2 = b2_ref[...]
    w3 = w3_ref[...]
    b3 = b3_ref[...]

    def chain(xj):
        h = jnp.maximum(
            jnp.dot(w1, xj, preferred_element_type=jnp.float32) + b1, 0.0)
        h = jnp.maximum(
            jnp.dot(w2, h, preferred_element_type=jnp.float32) + b2, 0.0)
        return jnp.maximum(
            jnp.dot(w3, h, preferred_element_type=jnp.float32) + b3, 0.0)

    def body(j2, acc):
        blk = x_ref[0, pl.ds(j2 * 8, 8), :]
        acc = jnp.maximum(acc, chain(blk[0:4]))
        return jnp.maximum(acc, chain(blk[4:8]))

    o_ref[0] = lax.fori_loop(
        0, NSAMPLE // 2, body, jnp.zeros((64, SBLK), jnp.float32))


def _mlp(h0, w1, b1, w2, b2, w3, b3):
    b_ = h0.shape[0]
    return pl.pallas_call(
        _mlp_body,
        grid=(b_, NPOINT // SBLK),
        in_specs=[
            pl.BlockSpec((1, 4 * NSAMPLE, SBLK), lambda b, s: (b, 0, s)),
            pl.BlockSpec((32, 4), lambda b, s: (0, 0)),
            pl.BlockSpec((32, 1), lambda b, s: (0, 0)),
            pl.BlockSpec((32, 32), lambda b, s: (0, 0)),
            pl.BlockSpec((32, 1), lambda b, s: (0, 0)),
            pl.BlockSpec((64, 32), lambda b, s: (0, 0)),
            pl.BlockSpec((64, 1), lambda b, s: (0, 0)),
        ],
        out_specs=pl.BlockSpec((1, 64, SBLK), lambda b, s: (b, 0, s)),
        out_shape=jax.ShapeDtypeStruct((b_, 64, NPOINT), jnp.float32),
    )(h0, w1, b1, w2, b2, w3, b3)


def kernel(xyz, features, W1, b1, W2, b2, W3, b3):
    b_, n, _ = xyz.shape
    xyz_t = jnp.transpose(xyz, (0, 2, 1))
    xyz_r = xyz_t.reshape(b_, 3, NROW, n // NROW)
    ox, oy, oz = _fps(xyz_r)
    ox, oy, oz = ox[:, 0], oy[:, 0], oz[:, 0]
    new_xyz = jnp.stack([ox, oy, oz], axis=-1)
    h0 = _make_ball_query(b_, n)(
        xyz_t[:, 0], xyz_t[:, 1], xyz_t[:, 2], features[:, 0], ox, oy, oz)
    nf_perm = _mlp(h0, W1, b1.reshape(32, 1), W2, b2.reshape(32, 1),
                   W3, b3.reshape(64, 1))
    wpb = 32 // b_
    new_features = (nf_perm.reshape(b_, 64, wpb, NPOINT // wpb)
                    .transpose(0, 1, 3, 2).reshape(b_, 64, NPOINT))
    return new_xyz, new_features


# speedup vs baseline: 19.0331x; 1.1896x over previous
import functools

import jax
import jax.numpy as jnp
from jax import lax
from jax.experimental import pallas as pl
from jax.experimental.pallas import tpu as pltpu
from jax.experimental.pallas import tpu_sc as plsc

NPOINT = 1024
NSAMPLE = 32
RADIUS2 = 1.0
NROW = 8
LANES = 16
SBLK = 512


def _fps_body(xyz_ref, ox_ref, oy_ref, oz_ref, dists_ref):
    b_ = xyz_ref.shape[0]
    nrow, ncol = xyz_ref.shape[2], xyz_ref.shape[3]
    n = nrow * ncol
    shape = (nrow, ncol)
    lin = (lax.broadcasted_iota(jnp.int32, shape, 0) * ncol
           + lax.broadcasted_iota(jnp.int32, shape, 1))
    xs = [xyz_ref[b, 0] for b in range(b_)]
    ys = [xyz_ref[b, 1] for b in range(b_)]
    zs = [xyz_ref[b, 2] for b in range(b_)]

    def body(i, fars):
        new_fars = []
        for b in range(b_):
            far = fars[b]
            x, y, z = xs[b], ys[b], zs[b]
            sel = lin == far
            cx = jnp.sum(jnp.where(sel, x, 0.0))
            cy = jnp.sum(jnp.where(sel, y, 0.0))
            cz = jnp.sum(jnp.where(sel, z, 0.0))
            ox_ref[b, 0, i] = cx
            oy_ref[b, 0, i] = cy
            oz_ref[b, 0, i] = cz
            dx = x - cx
            dy = y - cy
            dz = z - cz
            d = dx * dx + dy * dy + dz * dz
            nd = jnp.minimum(dists_ref[b], d)
            dists_ref[b] = nd
            m = jnp.max(nd)
            new_fars.append(
                jnp.min(jnp.where(nd == m, lin, n)).astype(jnp.int32))
        return tuple(new_fars)

    for b in range(b_):
        dists_ref[b] = jnp.full(shape, 1e10, jnp.float32)
    lax.fori_loop(0, NPOINT, body, (jnp.int32(0),) * b_)


def _fps(xyz_r):
    b_, _, nrow, ncol = xyz_r.shape
    out_shape = [jax.ShapeDtypeStruct((b_, 1, NPOINT), jnp.float32)] * 3
    return pl.pallas_call(
        _fps_body,
        grid=(1,),
        in_specs=[pl.BlockSpec((b_, 3, nrow, ncol), lambda g: (0, 0, 0, 0))],
        out_specs=[pl.BlockSpec((b_, 1, NPOINT), lambda g: (0, 0, 0),
                                memory_space=pltpu.SMEM)] * 3,
        out_shape=out_shape,
        scratch_shapes=[pltpu.VMEM((b_, nrow, ncol), jnp.float32)],
    )(xyz_r)


UNROLL = 4


def _make_ball_query(b_, n):
    nworkers = 32
    wpb = nworkers // b_
    cpw = NPOINT // wpb
    nchunk = n // LANES
    mesh = plsc.VectorSubcoreMesh(core_axis_name="c", subcore_axis_name="s")

    @functools.partial(
        pl.kernel,
        mesh=mesh,
        compiler_params=pltpu.CompilerParams(needs_layout_passes=False),
        out_type=jax.ShapeDtypeStruct((b_, 4 * NSAMPLE, NPOINT), jnp.float32),
        scratch_types=[
            pltpu.VMEM((n,), jnp.float32),
            pltpu.VMEM((n,), jnp.float32),
            pltpu.VMEM((n,), jnp.float32),
            pltpu.VMEM((n,), jnp.float32),
            pltpu.VMEM((NPOINT,), jnp.float32),
            pltpu.VMEM((NPOINT,), jnp.float32),
            pltpu.VMEM((NPOINT,), jnp.float32),
            pltpu.VMEM((128,), jnp.int32),
            pltpu.VMEM((4 * NSAMPLE, cpw), jnp.float32),
        ],
    )
    def bq(x_hbm, y_hbm, z_hbm, f_hbm, cx_hbm, cy_hbm, cz_hbm, out_hbm,
           x_v, y_v, z_v, f_v, cx_v, cy_v, cz_v, idx_v, out_v):
        wid = lax.axis_index("s") * 2 + lax.axis_index("c")
        b = wid // wpb
        k = wid % wpb
        pltpu.sync_copy(x_hbm.at[b], x_v)
        pltpu.sync_copy(y_hbm.at[b], y_v)
        pltpu.sync_copy(z_hbm.at[b], z_v)
        pltpu.sync_copy(f_hbm.at[b], f_v)
        pltpu.sync_copy(cx_hbm.at[b], cx_v)
        pltpu.sync_copy(cy_hbm.at[b], cy_v)
        pltpu.sync_copy(cz_hbm.at[b], cz_v)
        lanes = lax.iota(jnp.int32, LANES)

        def per_centroid(ci, carry):
            splat_s = jnp.full((LANES,), ci * wpb + k, jnp.int32)
            cxv = plsc.load_gather(cx_v, [splat_s])
            cyv = plsc.load_gather(cy_v, [splat_s])
            czv = plsc.load_gather(cz_v, [splat_s])

            def w_cond(st):
                chunk, count = st
                return jnp.logical_and(count < NSAMPLE, chunk < nchunk)

            def w_body(st):
                chunk, count = st
                ms = []
                packed = jnp.zeros((LANES,), jnp.int32)
                for u in range(UNROLL):
                    base = (chunk + u) * LANES
                    xv = x_v[pl.ds(base, LANES)]
                    yv = y_v[pl.ds(base, LANES)]
                    zv = z_v[pl.ds(base, LANES)]
                    dx = xv - cxv
                    dy = yv - cyv
                    dz = zv - czv
                    d = dx * dx + dy * dy + dz * dz
                    m = d <= RADIUS2
                    ms.append(m)
                    cnt_u = plsc.all_reduce_population_count(m)
                    packed = packed + (cnt_u << (8 * u))
                tot = jnp.max(packed)

                @pl.when(tot > 0)
                def _append():
                    off = count
                    for u in range(UNROLL):
                        plsc.store_compressed(
                            idx_v.at[pl.ds(off, LANES)],
                            (chunk + u) * LANES + lanes, mask=ms[u])
                        off = off + ((tot >> (8 * u)) & 0xFF)

                total = ((tot & 0xFF) + ((tot >> 8) & 0xFF)
                         + ((tot >> 16) & 0xFF) + ((tot >> 24) & 0xFF))
                return (chunk + UNROLL, count + total)

            _, count = lax.while_loop(w_cond, w_body,
                                      (jnp.int32(0), jnp.int32(0)))
            head = idx_v[pl.ds(0, LANES)]
            first_s = jnp.max(jnp.where(lanes == 0, head,
                                        jnp.int32(-2147483648)))
            first = jnp.full((LANES,), first_s, jnp.int32)
            countv = jnp.full((LANES,), count, jnp.int32)
            for h in range(NSAMPLE // LANES):
                slots = lanes + LANES * h
                cur = idx_v[pl.ds(LANES * h, LANES)]
                idx_v[pl.ds(LANES * h, LANES)] = jnp.where(
                    slots < countv, cur, first)
            splat_c = jnp.full((LANES,), ci, jnp.int32)
            for h in range(NSAMPLE // LANES):
                iv = jnp.clip(idx_v[pl.ds(LANES * h, LANES)], 0, n - 1)
                rows = (lanes + LANES * h) * 4
                gx = plsc.load_gather(x_v, [iv]) - cxv
                plsc.store_scatter(out_v, [rows, splat_c], gx)
                gy = plsc.load_gather(y_v, [iv]) - cyv
                plsc.store_scatter(out_v, [rows + 1, splat_c], gy)
                gz = plsc.load_gather(z_v, [iv]) - czv
                plsc.store_scatter(out_v, [rows + 2, splat_c], gz)
                gf = plsc.load_gather(f_v, [iv])
                plsc.store_scatter(out_v, [rows + 3, splat_c], gf)
            return carry

        lax.fori_loop(0, cpw, per_centroid, jnp.int32(0))
        pltpu.sync_copy(out_v, out_hbm.at[b, :, pl.ds(k * cpw, cpw)])

    return bq


def _mlp_body(x_ref, w1_ref, b1_ref, w2_ref, b2_ref, w3_ref, b3_ref, o_ref):
    w1 = w1_ref[...]
    b1 = b1_ref[...]
    w2 = w2_ref[...]
    b2 = b2_ref[...]
    w3 = w3_ref[...]
    b3 = b3_ref[...]

    def chain(xj):
        h = jnp.maximum(
            jnp.dot(w1, xj, preferred_element_type=jnp.float32) + b1, 0.0)
        h = jnp.maximum(
            jnp.dot(w2, h, preferred_element_type=jnp.float32) + b2, 0.0)
        return jnp.maximum(
            jnp.dot(w3, h, preferred_element_type=jnp.float32) + b3, 0.0)

    def body(j2, acc):
        blk = x_ref[0, pl.ds(j2 * 8, 8), :]
        acc = jnp.maximum(acc, chain(blk[0:4]))
        return jnp.maximum(acc, chain(blk[4:8]))

    o_ref[0] = lax.fori_loop(
        0, NSAMPLE // 2, body, jnp.zeros((64, SBLK), jnp.float32))


def _mlp(h0, w1, b1, w2, b2, w3, b3):
    b_ = h0.shape[0]
    return pl.pallas_call(
        _mlp_body,
        grid=(b_, NPOINT // SBLK),
        in_specs=[
            pl.BlockSpec((1, 4 * NSAMPLE, SBLK), lambda b, s: (b, 0, s)),
            pl.BlockSpec((32, 4), lambda b, s: (0, 0)),
            pl.BlockSpec((32, 1), lambda b, s: (0, 0)),
            pl.BlockSpec((32, 32), lambda b, s: (0, 0)),
            pl.BlockSpec((32, 1), lambda b, s: (0, 0)),
            pl.BlockSpec((64, 32), lambda b, s: (0, 0)),
            pl.BlockSpec((64, 1), lambda b, s: (0, 0)),
        ],
        out_specs=pl.BlockSpec((1, 64, SBLK), lambda b, s: (b, 0, s)),
        out_shape=jax.ShapeDtypeStruct((b_, 64, NPOINT), jnp.float32),
    )(h0, w1, b1, w2, b2, w3, b3)


def kernel(xyz, features, W1, b1, W2, b2, W3, b3):
    b_, n, _ = xyz.shape
    xyz_t = jnp.transpose(xyz, (0, 2, 1))
    xyz_r = xyz_t.reshape(b_, 3, NROW, n // NROW)
    ox, oy, oz = _fps(xyz_r)
    ox, oy, oz = ox[:, 0], oy[:, 0], oz[:, 0]
    new_xyz = jnp.stack([ox, oy, oz], axis=-1)
    h0 = _make_ball_query(b_, n)(
        xyz_t[:, 0], xyz_t[:, 1], xyz_t[:, 2], features[:, 0], ox, oy, oz)
    nf_perm = _mlp(h0, W1, b1.reshape(32, 1), W2, b2.reshape(32, 1),
                   W3, b3.reshape(64, 1))
    wpb = 32 // b_
    new_features = (nf_perm.reshape(b_, 64, wpb, NPOINT // wpb)
                    .transpose(0, 1, 3, 2).reshape(b_, 64, NPOINT))
    return new_xyz, new_features
